# Initial kernel scaffold; baseline (speedup 1.0000x reference)
import jax
import jax.numpy as jnp
from jax.experimental import pallas as pl


def kernel(tokens, prompt_table, W1, b1, W2, b2, W3, b3, token_table):
    raise NotImplementedError("write your pallas kernel here")


# speedup vs baseline: 1.7179x; 1.7179x over previous
import functools

import jax
import jax.numpy as jnp
from jax import lax
from jax.experimental import pallas as pl
from jax.experimental.pallas import tpu as pltpu
from jax.experimental.pallas import tpu_sc as plsc

PROMPT_LEN = 128
HIDDEN = 2048
BATCH = 4
SEQ = 2048

NC = 2
NS = 16
NW = NC * NS

TOK_TOTAL = BATCH * SEQ
ROWS_PER_W = TOK_TOTAL // NW
CHUNK = 16
NCHUNK = ROWS_PER_W // CHUNK
SEQ_PER_W = SEQ // (NW // BATCH)
P_PER_W = PROMPT_LEN // (NW // BATCH)


def _linear_body(x_ref, w_ref, b_ref, o_ref, *, relu):
    acc = jnp.dot(x_ref[...], w_ref[...], preferred_element_type=jnp.float32)
    acc = acc + b_ref[...]
    if relu:
        acc = jnp.maximum(acc, 0.0)
    o_ref[...] = acc


def _linear(x, w, b, relu, bn=512):
    n = w.shape[1]
    grid = n // bn
    return pl.pallas_call(
        functools.partial(_linear_body, relu=relu),
        grid=(grid,),
        in_specs=[
            pl.BlockSpec((x.shape[0], x.shape[1]), lambda j: (0, 0)),
            pl.BlockSpec((w.shape[0], bn), lambda j: (0, j)),
            pl.BlockSpec((1, bn), lambda j: (0, j)),
        ],
        out_specs=pl.BlockSpec((x.shape[0], bn), lambda j: (0, j)),
        out_shape=jax.ShapeDtypeStruct((x.shape[0], n), jnp.float32),
    )(x, w, b.reshape(1, -1))


_MESH = plsc.VectorSubcoreMesh(core_axis_name="c", subcore_axis_name="s")


@functools.partial(
    pl.kernel,
    out_type=jax.ShapeDtypeStruct((BATCH, PROMPT_LEN + SEQ, HIDDEN),
                                  jnp.float32),
    mesh=_MESH,
    scratch_types=[
        pltpu.VMEM((ROWS_PER_W,), jnp.int32),
        pltpu.VMEM((CHUNK, HIDDEN), jnp.float32),
        pltpu.VMEM((CHUNK, HIDDEN), jnp.float32),
        pltpu.SemaphoreType.DMA,
        pltpu.SemaphoreType.DMA,
        pltpu.SemaphoreType.DMA,
        pltpu.SemaphoreType.DMA,
    ],
)
def _sc_assemble(tokens_hbm, pe_hbm, table_hbm, out_hbm,
                 idx_v, buf0, buf1, gsem0, gsem1, wsem0, wsem1):
    wid = lax.axis_index("s") * NC + lax.axis_index("c")
    wpb = NW // BATCH
    b = wid // wpb
    s0 = (wid % wpb) * SEQ_PER_W
    p0 = (wid % wpb) * P_PER_W

    pltpu.sync_copy(tokens_hbm.at[b, pl.ds(s0, ROWS_PER_W)], idx_v)

    pltpu.sync_copy(pe_hbm.at[pl.ds(p0, P_PER_W)], buf0.at[pl.ds(0, P_PER_W)])
    pltpu.sync_copy(buf0.at[pl.ds(0, P_PER_W)], out_hbm.at[b, pl.ds(p0, P_PER_W)])

    bufs = (buf0, buf1)
    gsems = (gsem0, gsem1)
    wsems = (wsem0, wsem1)

    def start_gather(c):
        i = c % 2
        return pltpu.async_copy(
            table_hbm.at[idx_v.at[pl.ds(c * CHUNK, CHUNK)]], bufs[i], gsems[i])

    def start_write(c):
        i = c % 2
        return pltpu.async_copy(
            bufs[i], out_hbm.at[b, pl.ds(PROMPT_LEN + s0 + c * CHUNK, CHUNK)],
            wsems[i])

    writes = [None, None]
    g = start_gather(0)
    for c in range(NCHUNK):
        i = c % 2
        ni = (c + 1) % 2
        gn = None
        if c + 1 < NCHUNK:
            if writes[ni] is not None:
                writes[ni].wait()
                writes[ni] = None
            gn = start_gather(c + 1)
        g.wait()
        writes[i] = start_write(c)
        g = gn
    for i in range(2):
        if writes[i] is not None:
            writes[i].wait()


def kernel(tokens, prompt_table, W1, b1, W2, b2, W3, b3, token_table):
    h = _linear(prompt_table, W1, b1, relu=True)
    h = _linear(h, W2, b2, relu=True)
    pe = _linear(h, W3, b3, relu=False)
    return _sc_assemble(tokens.astype(jnp.int32), pe, token_table)


# speedup vs baseline: 1.9430x; 1.1310x over previous
import functools

import jax
import jax.numpy as jnp
from jax import lax
from jax.experimental import pallas as pl
from jax.experimental.pallas import tpu as pltpu
from jax.experimental.pallas import tpu_sc as plsc

PROMPT_LEN = 128
HIDDEN = 2048
BATCH = 4
SEQ = 2048
TOTAL = PROMPT_LEN + SEQ

NC = 2
NS = 16
NW = NC * NS

TOK_TOTAL = BATCH * SEQ
ROWS_PER_W = TOK_TOTAL // NW
CHUNK = 16
NCHUNK = ROWS_PER_W // CHUNK
WPB = NW // BATCH
SEQ_PER_W = SEQ // WPB


def _linear_body(x_ref, w_ref, b_ref, o_ref, *, relu):
    acc = jnp.dot(x_ref[...], w_ref[...], preferred_element_type=jnp.float32)
    acc = acc + b_ref[...]
    if relu:
        acc = jnp.maximum(acc, 0.0)
    o_ref[...] = acc


def _linear(x, w, b, relu, bn=512):
    n = w.shape[1]
    grid = n // bn
    return pl.pallas_call(
        functools.partial(_linear_body, relu=relu),
        grid=(grid,),
        in_specs=[
            pl.BlockSpec((x.shape[0], x.shape[1]), lambda j: (0, 0)),
            pl.BlockSpec((w.shape[0], bn), lambda j: (0, j)),
            pl.BlockSpec((1, bn), lambda j: (0, j)),
        ],
        out_specs=pl.BlockSpec((x.shape[0], bn), lambda j: (0, j)),
        out_shape=jax.ShapeDtypeStruct((x.shape[0], n), jnp.float32),
    )(x, w, b.reshape(1, -1))


def _final_linear_body(x_ref, w_ref, b_ref, _, o_ref):
    acc = jnp.dot(x_ref[...], w_ref[...], preferred_element_type=jnp.float32)
    acc = acc + b_ref[...]
    o_ref[...] = jnp.broadcast_to(acc[None, :, :], o_ref.shape)


def _final_linear_into(x, w, b, out_buf, bn=512):
    grid = HIDDEN // bn
    return pl.pallas_call(
        _final_linear_body,
        grid=(grid,),
        in_specs=[
            pl.BlockSpec((PROMPT_LEN, HIDDEN), lambda j: (0, 0)),
            pl.BlockSpec((HIDDEN, bn), lambda j: (0, j)),
            pl.BlockSpec((1, bn), lambda j: (0, j)),
            pl.BlockSpec(memory_space=pl.ANY),
        ],
        out_specs=pl.BlockSpec((BATCH, PROMPT_LEN, bn), lambda j: (0, 0, j)),
        out_shape=jax.ShapeDtypeStruct((BATCH, TOTAL, HIDDEN), jnp.float32),
        input_output_aliases={3: 0},
    )(x, w, b.reshape(1, -1), out_buf)


_MESH = plsc.VectorSubcoreMesh(core_axis_name="c", subcore_axis_name="s")


@functools.partial(
    pl.kernel,
    out_type=jax.ShapeDtypeStruct((BATCH, TOTAL, HIDDEN), jnp.float32),
    mesh=_MESH,
    scratch_types=[
        pltpu.VMEM((ROWS_PER_W,), jnp.int32),
        pltpu.VMEM((CHUNK, HIDDEN), jnp.float32),
        pltpu.VMEM((CHUNK, HIDDEN), jnp.float32),
        pltpu.SemaphoreType.DMA,
        pltpu.SemaphoreType.DMA,
        pltpu.SemaphoreType.DMA,
        pltpu.SemaphoreType.DMA,
    ],
)
def _sc_gather(tokens_hbm, table_hbm, out_hbm,
               idx_v, buf0, buf1, gsem0, gsem1, wsem0, wsem1):
    wid = lax.axis_index("s") * NC + lax.axis_index("c")
    b = wid // WPB
    s0 = (wid % WPB) * SEQ_PER_W

    pltpu.sync_copy(tokens_hbm.at[b, pl.ds(s0, ROWS_PER_W)], idx_v)

    bufs = (buf0, buf1)
    gsems = (gsem0, gsem1)
    wsems = (wsem0, wsem1)

    def start_gather(c):
        i = c % 2
        return pltpu.async_copy(
            table_hbm.at[idx_v.at[pl.ds(c * CHUNK, CHUNK)]], bufs[i], gsems[i])

    def start_write(c):
        i = c % 2
        return pltpu.async_copy(
            bufs[i], out_hbm.at[b, pl.ds(PROMPT_LEN + s0 + c * CHUNK, CHUNK)],
            wsems[i])

    writes = [None, None]
    g = start_gather(0)
    for c in range(NCHUNK):
        i = c % 2
        ni = (c + 1) % 2
        gn = None
        if c + 1 < NCHUNK:
            if writes[ni] is not None:
                writes[ni].wait()
                writes[ni] = None
            gn = start_gather(c + 1)
        g.wait()
        writes[i] = start_write(c)
        g = gn
    for i in range(2):
        if writes[i] is not None:
            writes[i].wait()


def kernel(tokens, prompt_table, W1, b1, W2, b2, W3, b3, token_table):
    out = _sc_gather(tokens.astype(jnp.int32), token_table)
    h = _linear(prompt_table, W1, b1, relu=True)
    h = _linear(h, W2, b2, relu=True)
    return _final_linear_into(h, W3, b3, out)


# speedup vs baseline: 1.9662x; 1.0120x over previous
import functools

import jax
import jax.numpy as jnp
from jax import lax
from jax.experimental import pallas as pl
from jax.experimental.pallas import tpu as pltpu
from jax.experimental.pallas import tpu_sc as plsc

PROMPT_LEN = 128
HIDDEN = 2048
BATCH = 4
SEQ = 2048
TOTAL = PROMPT_LEN + SEQ

NC = 2
NS = 16
NW = NC * NS

TOK_TOTAL = BATCH * SEQ
ROWS_PER_W = TOK_TOTAL // NW
CHUNK = 16
NCHUNK = ROWS_PER_W // CHUNK
WPB = NW // BATCH
SEQ_PER_W = SEQ // WPB


def _linear_body(x_ref, w_ref, b_ref, o_ref, *, relu):
    acc = jnp.dot(x_ref[...], w_ref[...], preferred_element_type=jnp.float32)
    acc = acc + b_ref[...]
    if relu:
        acc = jnp.maximum(acc, 0.0)
    o_ref[...] = acc


def _linear(x, w, b, relu, bn=512):
    n = w.shape[1]
    grid = n // bn
    return pl.pallas_call(
        functools.partial(_linear_body, relu=relu),
        grid=(grid,),
        in_specs=[
            pl.BlockSpec((x.shape[0], x.shape[1]), lambda j: (0, 0)),
            pl.BlockSpec((w.shape[0], bn), lambda j: (0, j)),
            pl.BlockSpec((1, bn), lambda j: (0, j)),
        ],
        out_specs=pl.BlockSpec((x.shape[0], bn), lambda j: (0, j)),
        out_shape=jax.ShapeDtypeStruct((x.shape[0], n), jnp.float32),
    )(x, w, b.reshape(1, -1))


def _broadcast_body(pe_ref, _, o_ref):
    o_ref[...] = jnp.broadcast_to(pe_ref[...][None, :, :], o_ref.shape)


def _broadcast_into(pe, out_buf, bn=1024):
    grid = HIDDEN // bn
    return pl.pallas_call(
        _broadcast_body,
        grid=(grid,),
        in_specs=[
            pl.BlockSpec((PROMPT_LEN, bn), lambda j: (0, j)),
            pl.BlockSpec(memory_space=pl.ANY),
        ],
        out_specs=pl.BlockSpec((BATCH, PROMPT_LEN, bn), lambda j: (0, 0, j)),
        out_shape=jax.ShapeDtypeStruct((BATCH, TOTAL, HIDDEN), jnp.float32),
        input_output_aliases={1: 0},
    )(pe, out_buf)


_MESH = plsc.VectorSubcoreMesh(core_axis_name="c", subcore_axis_name="s")


@functools.partial(
    pl.kernel,
    out_type=jax.ShapeDtypeStruct((BATCH, TOTAL, HIDDEN), jnp.float32),
    mesh=_MESH,
    scratch_types=[
        pltpu.VMEM((ROWS_PER_W,), jnp.int32),
        pltpu.VMEM((CHUNK, HIDDEN), jnp.float32),
        pltpu.VMEM((CHUNK, HIDDEN), jnp.float32),
        pltpu.VMEM((CHUNK, HIDDEN), jnp.float32),
        pltpu.SemaphoreType.DMA,
        pltpu.SemaphoreType.DMA,
        pltpu.SemaphoreType.DMA,
        pltpu.SemaphoreType.DMA,
        pltpu.SemaphoreType.DMA,
        pltpu.SemaphoreType.DMA,
    ],
)
def _sc_gather(tokens_hbm, table_hbm, out_hbm,
               idx_v, buf0, buf1, buf2,
               gsem0, gsem1, gsem2, wsem0, wsem1, wsem2):
    wid = lax.axis_index("s") * NC + lax.axis_index("c")
    b = wid // WPB
    s0 = (wid % WPB) * SEQ_PER_W

    pltpu.sync_copy(tokens_hbm.at[b, pl.ds(s0, ROWS_PER_W)], idx_v)

    NB = 3
    bufs = (buf0, buf1, buf2)
    gsems = (gsem0, gsem1, gsem2)
    wsems = (wsem0, wsem1, wsem2)

    def start_gather(c):
        i = c % NB
        return pltpu.async_copy(
            table_hbm.at[idx_v.at[pl.ds(c * CHUNK, CHUNK)]], bufs[i], gsems[i])

    def start_write(c):
        i = c % NB
        return pltpu.async_copy(
            bufs[i], out_hbm.at[b, pl.ds(PROMPT_LEN + s0 + c * CHUNK, CHUNK)],
            wsems[i])

    writes = [None] * NB
    g = [None] * NB
    for c in range(min(2, NCHUNK)):
        g[c % NB] = start_gather(c)
    for c in range(NCHUNK):
        i = c % NB
        g[i].wait()
        g[i] = None
        writes[i] = start_write(c)
        nc = c + 2
        if nc < NCHUNK:
            j = nc % NB
            if writes[j] is not None:
                writes[j].wait()
                writes[j] = None
            g[j] = start_gather(nc)
    for i in range(NB):
        if writes[i] is not None:
            writes[i].wait()


def kernel(tokens, prompt_table, W1, b1, W2, b2, W3, b3, token_table):
    out = _sc_gather(tokens.astype(jnp.int32), token_table)
    h = _linear(prompt_table, W1, b1, relu=True)
    h = _linear(h, W2, b2, relu=True)
    pe = _linear(h, W3, b3, relu=False)
    return _broadcast_into(pe, out)


# speedup vs baseline: 1.9819x; 1.0080x over previous
import functools

import jax
import jax.numpy as jnp
from jax import lax
from jax.experimental import pallas as pl
from jax.experimental.pallas import tpu as pltpu
from jax.experimental.pallas import tpu_sc as plsc

PROMPT_LEN = 128
HIDDEN = 2048
BATCH = 4
SEQ = 2048
TOTAL = PROMPT_LEN + SEQ

NC = 2
NS = 16
NW = NC * NS

TOK_TOTAL = BATCH * SEQ
ROWS_PER_W = TOK_TOTAL // NW
CHUNK = 16
NCHUNK = ROWS_PER_W // CHUNK
WPB = NW // BATCH
SEQ_PER_W = SEQ // WPB


def _linear_body(x_ref, w_ref, b_ref, o_ref, *, relu):
    acc = jnp.dot(x_ref[...], w_ref[...], preferred_element_type=jnp.float32)
    acc = acc + b_ref[...]
    if relu:
        acc = jnp.maximum(acc, 0.0)
    o_ref[...] = acc


def _linear(x, w, b, relu, bn=1024):
    n = w.shape[1]
    grid = n // bn
    return pl.pallas_call(
        functools.partial(_linear_body, relu=relu),
        grid=(grid,),
        in_specs=[
            pl.BlockSpec((x.shape[0], x.shape[1]), lambda j: (0, 0)),
            pl.BlockSpec((w.shape[0], bn), lambda j: (0, j)),
            pl.BlockSpec((1, bn), lambda j: (0, j)),
        ],
        out_specs=pl.BlockSpec((x.shape[0], bn), lambda j: (0, j)),
        out_shape=jax.ShapeDtypeStruct((x.shape[0], n), jnp.float32),
    )(x, w, b.reshape(1, -1))


def _broadcast_body(pe_ref, _, o_ref):
    o_ref[...] = jnp.broadcast_to(pe_ref[...][None, :, :], o_ref.shape)


def _broadcast_into(pe, out_buf, bn=1024):
    grid = HIDDEN // bn
    return pl.pallas_call(
        _broadcast_body,
        grid=(grid,),
        in_specs=[
            pl.BlockSpec((PROMPT_LEN, bn), lambda j: (0, j)),
            pl.BlockSpec(memory_space=pl.ANY),
        ],
        out_specs=pl.BlockSpec((BATCH, PROMPT_LEN, bn), lambda j: (0, 0, j)),
        out_shape=jax.ShapeDtypeStruct((BATCH, TOTAL, HIDDEN), jnp.float32),
        input_output_aliases={1: 0},
    )(pe, out_buf)


_MESH = plsc.VectorSubcoreMesh(core_axis_name="c", subcore_axis_name="s")


@functools.partial(
    pl.kernel,
    out_type=jax.ShapeDtypeStruct((BATCH, TOTAL, HIDDEN), jnp.float32),
    mesh=_MESH,
    scratch_types=[
        pltpu.VMEM((ROWS_PER_W,), jnp.int32),
        pltpu.VMEM((CHUNK, HIDDEN), jnp.float32),
        pltpu.VMEM((CHUNK, HIDDEN), jnp.float32),
        pltpu.VMEM((CHUNK, HIDDEN), jnp.float32),
        pltpu.SemaphoreType.DMA,
        pltpu.SemaphoreType.DMA,
        pltpu.SemaphoreType.DMA,
        pltpu.SemaphoreType.DMA,
        pltpu.SemaphoreType.DMA,
        pltpu.SemaphoreType.DMA,
    ],
)
def _sc_gather(tokens_hbm, table_hbm, out_hbm,
               idx_v, buf0, buf1, buf2,
               gsem0, gsem1, gsem2, wsem0, wsem1, wsem2):
    wid = lax.axis_index("s") * NC + lax.axis_index("c")
    b = wid // WPB
    s0 = (wid % WPB) * SEQ_PER_W

    pltpu.sync_copy(tokens_hbm.at[b, pl.ds(s0, ROWS_PER_W)], idx_v)

    NB = 3
    bufs = (buf0, buf1, buf2)
    gsems = (gsem0, gsem1, gsem2)
    wsems = (wsem0, wsem1, wsem2)

    def start_gather(c):
        i = c % NB
        return pltpu.async_copy(
            table_hbm.at[idx_v.at[pl.ds(c * CHUNK, CHUNK)]], bufs[i], gsems[i])

    def start_write(c):
        i = c % NB
        return pltpu.async_copy(
            bufs[i], out_hbm.at[b, pl.ds(PROMPT_LEN + s0 + c * CHUNK, CHUNK)],
            wsems[i])

    writes = [None] * NB
    g = [None] * NB
    for c in range(min(2, NCHUNK)):
        g[c % NB] = start_gather(c)
    for c in range(NCHUNK):
        i = c % NB
        g[i].wait()
        g[i] = None
        writes[i] = start_write(c)
        nc = c + 2
        if nc < NCHUNK:
            j = nc % NB
            if writes[j] is not None:
                writes[j].wait()
                writes[j] = None
            g[j] = start_gather(nc)
    for i in range(NB):
        if writes[i] is not None:
            writes[i].wait()


def kernel(tokens, prompt_table, W1, b1, W2, b2, W3, b3, token_table):
    out = _sc_gather(tokens.astype(jnp.int32), token_table)
    h = _linear(prompt_table, W1, b1, relu=True)
    h = _linear(h, W2, b2, relu=True)
    pe = _linear(h, W3, b3, relu=False)
    return _broadcast_into(pe, out)
